# Initial kernel scaffold; baseline (speedup 1.0000x reference)
import jax
import jax.numpy as jnp
from jax.experimental import pallas as pl


def kernel(edge_index, h, W1, b1, W2, b2, lin1_W, lin1_b, lin2_W, lin2_b):
    raise NotImplementedError("write your pallas kernel here")


# speedup vs baseline: 10.2200x; 10.2200x over previous
import functools

import jax
import jax.numpy as jnp
from jax import lax
from jax.experimental import pallas as pl
from jax.experimental.pallas import tpu as pltpu
from jax.experimental.pallas import tpu_sc as plsc

N_NODES = 10000
N_EDGES = 320000
D = 128
N_PATH = 6
PATH_LEN = 20

NC = 2
NS = 16
NW = NC * NS
EDGES_PER_W = N_EDGES // NW
CH = 100
NCHUNK = EDGES_PER_W // CH
K = 1
NGRP = NCHUNK // (2 * K)
ZROWS = 2 * K * CH
R0 = 624


def _seg_sum_sc(idx4, feat):
    mesh = plsc.VectorSubcoreMesh(core_axis_name="c", subcore_axis_name="s",
                                  num_cores=NC, num_subcores=NS)

    @functools.partial(
        pl.kernel,
        out_type=jax.ShapeDtypeStruct((NC, N_NODES, D), jnp.float32),
        mesh=mesh,
        compiler_params=pltpu.CompilerParams(use_tc_tiling_on_sc=False),
        scratch_types=[
            pltpu.VMEM((NCHUNK, CH), jnp.int32),
            pltpu.VMEM((NCHUNK, CH), jnp.int32),
            pltpu.VMEM((ZROWS, D), jnp.float32),
            pltpu.VMEM_SHARED((N_NODES, D), jnp.float32),
            pltpu.SemaphoreType.DMA,
            pltpu.SemaphoreType.DMA,
            pltpu.SemaphoreType.DMA,
            pltpu.SemaphoreType.DMA,
        ],
    )
    def k(idx_hbm, feat_hbm, out_hbm, srcs_v, dsts_v, gbufs,
          acc_sh, gsemA, gsemB, ssemA, ssemB):
        c = lax.axis_index("c")
        s = lax.axis_index("s")
        wid = c * NS + s

        pltpu.sync_copy(idx_hbm.at[0, wid], srcs_v)
        pltpu.sync_copy(idx_hbm.at[1, wid], dsts_v)

        def zrow(i, carry):
            for kk in range(D // 16):
                gbufs[i, pl.ds(kk * 16, 16)] = jnp.zeros((16,), jnp.float32)
            return carry

        lax.fori_loop(0, ZROWS, zrow, 0)
        for t in range(R0 // ZROWS):
            pltpu.sync_copy(gbufs.at[pl.ds(0, ZROWS)],
                            acc_sh.at[pl.ds(s * R0 + t * ZROWS, ZROWS)])
        rem = R0 % ZROWS
        if rem:
            pltpu.sync_copy(gbufs.at[pl.ds(0, rem)],
                            acc_sh.at[pl.ds(s * R0 + (R0 // ZROWS) * ZROWS,
                                            rem)])

        @pl.when(s == NS - 1)
        def _zero_tail():
            pltpu.sync_copy(gbufs.at[pl.ds(0, N_NODES - NS * R0)],
                            acc_sh.at[pl.ds(NS * R0, N_NODES - NS * R0)])

        plsc.subcore_barrier()

        def buf(b):
            return gbufs.at[pl.ds(b * CH, CH)]

        def gather(j, b, sem):
            return pltpu.async_copy(feat_hbm.at[srcs_v.at[j]], buf(b), sem)

        def scatter(j, b, sem):
            return pltpu.async_copy(buf(b), acc_sh.at[dsts_v.at[j]], sem,
                                    add=True)

        def gather_wait(b, sem):
            pltpu.make_async_copy(feat_hbm.at[srcs_v.at[0]], buf(b), sem).wait()

        def scatter_wait(b, sem):
            pltpu.make_async_copy(buf(b), acc_sh.at[dsts_v.at[0]], sem).wait()

        for b in range(K):
            gather(b, b, gsemA)

        def body(g, carry):
            base = g * 2 * K
            for b in range(K):
                gather(base + K + b, K + b, gsemB)
            for b in range(K):
                gather_wait(b, gsemA)
            for b in range(K):
                scatter(base + b, b, ssemA)
            for b in range(K):
                gather_wait(K + b, gsemB)
            for b in range(K):
                scatter(base + K + b, K + b, ssemB)
            for b in range(K):
                scatter_wait(b, ssemA)

            @pl.when(g < NGRP - 1)
            def _prefetch_next():
                for b in range(K):
                    gather(base + 2 * K + b, b, gsemA)

            for b in range(K):
                scatter_wait(K + b, ssemB)
            return carry

        lax.fori_loop(0, NGRP, body, 0)
        plsc.subcore_barrier()

        pltpu.sync_copy(acc_sh.at[pl.ds(s * R0, R0)],
                        out_hbm.at[c, pl.ds(s * R0, R0)])

        @pl.when(s == NS - 1)
        def _write_tail():
            pltpu.sync_copy(acc_sh.at[pl.ds(NS * R0, N_NODES - NS * R0)],
                            out_hbm.at[c, pl.ds(NS * R0, N_NODES - NS * R0)])

    return k(idx4, feat)


BLK = 2000


def _gin_tc(feat, p, W, b):
    nblk = N_NODES // BLK

    def body(h_ref, p_ref, w_ref, b_ref, o_ref):
        x = h_ref[...] + p_ref[0] + p_ref[1]
        y = jnp.dot(x, w_ref[...], preferred_element_type=jnp.float32)
        o_ref[...] = jnp.maximum(y + b_ref[...], 0.0)

    return pl.pallas_call(
        body,
        grid=(nblk,),
        in_specs=[
            pl.BlockSpec((BLK, D), lambda i: (i, 0)),
            pl.BlockSpec((NC, BLK, D), lambda i: (0, i, 0)),
            pl.BlockSpec((D, D), lambda i: (0, 0)),
            pl.BlockSpec((1, D), lambda i: (0, 0)),
        ],
        out_specs=pl.BlockSpec((BLK, D), lambda i: (i, 0)),
        out_shape=jax.ShapeDtypeStruct((N_NODES, D), jnp.float32),
    )(feat, p, W, b.reshape(1, D))


def _gin2_head_tc(feat, p, W, b, w1r, b1r, w2r, b2r):
    nblk = N_NODES // BLK

    def body(h_ref, p_ref, w_ref, b_ref, w1_ref, b1_ref, w2_ref,
             b2_ref, o_ref, colsum_sc, psum_sc):
        i = pl.program_id(0)
        x = h_ref[...] + p_ref[0] + p_ref[1]
        y = jnp.maximum(
            jnp.dot(x, w_ref[...], preferred_element_type=jnp.float32)
            + b_ref[...], 0.0)
        cs = jnp.sum(y, axis=0, keepdims=True)

        @pl.when(i == 0)
        def _first():
            colsum_sc[...] = cs
            psum_sc[...] = jnp.concatenate(
                [jnp.sum(y[p * PATH_LEN:(p + 1) * PATH_LEN, :], axis=0,
                         keepdims=True) for p in range(N_PATH)], axis=0)

        @pl.when(i > 0)
        def _rest():
            colsum_sc[...] += cs

        @pl.when(i == nblk - 1)
        def _finish():
            gmean = colsum_sc[...] * (1.0 / N_NODES)
            wm = w1_ref[0:1, :]
            ws = w1_ref[1:2, :]
            sv = (jnp.sum(gmean * wm, axis=1, keepdims=True)
                  + jnp.sum(psum_sc[...] * ws, axis=1, keepdims=True)
                  + b1_ref[0, 0])
            sv = jnp.tanh(sv)
            z = (jnp.sum(sv * w2_ref[...], axis=0, keepdims=True)
                 + b2_ref[...])
            zm = z - jnp.max(z, axis=1, keepdims=True)
            e = jnp.exp(zm)
            o_ref[...] = e / jnp.sum(e, axis=1, keepdims=True)

    zero2 = lambda i: (0, 0)
    return pl.pallas_call(
        body,
        grid=(nblk,),
        in_specs=[
            pl.BlockSpec((BLK, D), lambda i: (i, 0)),
            pl.BlockSpec((NC, BLK, D), lambda i: (0, i, 0)),
            pl.BlockSpec((D, D), zero2),
            pl.BlockSpec((1, D), zero2),
            pl.BlockSpec((2, D), zero2),
            pl.BlockSpec((1, 1), zero2),
            pl.BlockSpec((N_PATH, 2), zero2),
            pl.BlockSpec((1, 2), zero2),
        ],
        out_specs=pl.BlockSpec((1, 2), zero2),
        out_shape=jax.ShapeDtypeStruct((1, 2), jnp.float32),
        scratch_shapes=[
            pltpu.VMEM((1, D), jnp.float32),
            pltpu.VMEM((N_PATH, D), jnp.float32),
        ],
    )(feat, p, W, b.reshape(1, D), w1r, b1r, w2r, b2r)


def kernel(edge_index, h, W1, b1, W2, b2, lin1_W, lin1_b, lin2_W, lin2_b):
    idx4 = edge_index.reshape(2, NW, NCHUNK, CH)
    p1 = _seg_sum_sc(idx4, h)
    h1 = _gin_tc(h, p1, W1, b1)
    p2 = _seg_sum_sc(idx4, h1)
    w1r = lin1_W[:, 0].reshape(2, D)
    return _gin2_head_tc(h1, p2, W2, b2, w1r,
                         lin1_b.reshape(1, 1), lin2_W, lin2_b.reshape(1, 2))
